# baseline (speedup 1.0000x reference)
import functools

import jax
import jax.numpy as jnp
from jax.experimental import pallas as pl
from jax.experimental.pallas import tpu as pltpu

B, N, D = 128, 193, 512
K = 96
H = 1024
O = 2048
BB = 8
NBLK = B // BB
M = B * K

_HI = jax.lax.Precision.HIGHEST
_MED = jax.lax.Precision.DEFAULT


def _k1_body(scores_ref, x_ref, w1_ref, b1_ref, dlpw_ref, dlpb_ref,
             bf_ref, newbase_ref, c_ref, s_ref):
    i = pl.program_id(0)
    s = scores_ref[...]
    col = jax.lax.broadcasted_iota(jnp.int32, (BB, N), 1)
    s = jnp.where(col == 0, -1.0, s)

    si = s[:, :, None]
    sj = s[:, None, :]
    ii = jax.lax.broadcasted_iota(jnp.int32, (N, N), 0)[None]
    jj = jax.lax.broadcasted_iota(jnp.int32, (N, N), 1)[None]
    cmp = (sj > si) | ((sj == si) & (jj < ii))
    rank = jnp.sum(cmp.astype(jnp.float32), axis=2)

    r_iota = jax.lax.broadcasted_iota(jnp.int32, (BB, N, K), 2).astype(jnp.float32)
    p = (rank[:, :, None] == r_iota).astype(jnp.float32)

    x = x_ref[...]
    gathered = jax.lax.dot_general(
        p, x, (((1,), (1,)), ((0,), (0,))),
        preferred_element_type=jnp.float32, precision=_MED)

    w1v = w1_ref[...].reshape(1, 1, D)
    xs = jnp.sum(gathered * w1v, axis=2) + b1_ref[...]
    new = jax.lax.dot_general(
        xs, dlpw_ref[...], (((1,), (1,)), ((), ())),
        preferred_element_type=jnp.float32, precision=_MED) + dlpb_ref[...]
    nb = new * (1.0 / (jnp.sqrt(jnp.sum(new * new, axis=1, keepdims=True)) + 1e-8))
    newbase_ref[...] = nb

    sq = jnp.sum(gathered * gathered, axis=2, keepdims=True)
    bf = gathered * (1.0 / (jnp.sqrt(sq) + 1e-8))
    bf2 = bf.reshape(BB * K, D)
    bf_ref[...] = bf2

    bf2h = bf2.astype(jnp.bfloat16)
    cblk = jax.lax.dot_general(
        bf2h, bf2h, (((0,), (0,)), ((), ())),
        preferred_element_type=jnp.float32, precision=_MED)
    sblk = jnp.sum(bf2, axis=0, keepdims=True)

    @pl.when(i == 0)
    def _():
        c_ref[...] = cblk
        s_ref[...] = sblk

    @pl.when(i != 0)
    def _():
        c_ref[...] += cblk
        s_ref[...] += sblk


def _k2_body(bf_ref, newbase_ref, c_ref, s_ref, w0_ref, w0b_ref, b0_ref, g0_ref,
             beta0_ref, w1b_ref, b1_ref, out_ref, scale_ref, shift_ref):
    i = pl.program_id(0)

    @pl.when(i == 0)
    def _():
        w0 = w0_ref[...]
        sbar = s_ref[...] * (1.0 / M)
        m1 = jax.lax.dot_general(
            sbar, w0, (((1,), (1,)), ((), ())),
            preferred_element_type=jnp.float32, precision=_HI)
        w0c = jax.lax.dot_general(
            w0, c_ref[...], (((1,), (0,)), ((), ())),
            preferred_element_type=jnp.float32, precision=_HI)
        q = jnp.sum(w0c * w0, axis=1, keepdims=True).reshape(1, H) * (1.0 / M)
        b0 = b0_ref[...]
        mu = m1 + b0
        eh2 = q + 2.0 * b0 * m1 + b0 * b0
        var = eh2 - mu * mu
        sc = g0_ref[...] * jax.lax.rsqrt(var + 1e-5)
        scale_ref[...] = sc
        shift_ref[...] = (b0 - mu) * sc + beta0_ref[...]

    bfb = bf_ref[...].astype(jnp.bfloat16)
    h = jax.lax.dot_general(
        bfb, w0b_ref[...], (((1,), (1,)), ((), ())),
        preferred_element_type=jnp.float32, precision=_MED)
    a = jnp.maximum(h * scale_ref[...] + shift_ref[...], 0.0).astype(jnp.bfloat16)
    out = jax.lax.dot_general(
        a, w1b_ref[...], (((1,), (1,)), ((), ())),
        preferred_element_type=jnp.float32, precision=_MED)
    pooled = jnp.max(out.reshape(BB, K, O), axis=1)
    out_ref[...] = pooled + b1_ref[...] + newbase_ref[...]


@functools.partial(jax.jit, static_argnames=())
def kernel(base_features, atten, pid, weight_1, bias_1, dlp_lin1_w, dlp_lin1_b,
           mlp_w0, mlp_b0, mlp_g0, mlp_beta0, mlp_w1, mlp_b1):
    del pid
    scores = atten[:, 0, :]

    b1 = bias_1.reshape(1, 1)
    dlpb = dlp_lin1_b.reshape(1, O)
    b0 = mlp_b0.reshape(1, H)
    g0 = mlp_g0.reshape(1, H)
    beta0 = mlp_beta0.reshape(1, H)
    b1v = mlp_b1.reshape(1, O)

    bf, newbase, c_mat, s_vec = pl.pallas_call(
        _k1_body,
        grid=(NBLK,),
        in_specs=[
            pl.BlockSpec((BB, N), lambda i: (i, 0)),
            pl.BlockSpec((BB, N, D), lambda i: (i, 0, 0)),
            pl.BlockSpec((1, D), lambda i: (0, 0)),
            pl.BlockSpec((1, 1), lambda i: (0, 0)),
            pl.BlockSpec((O, K), lambda i: (0, 0)),
            pl.BlockSpec((1, O), lambda i: (0, 0)),
        ],
        out_specs=[
            pl.BlockSpec((BB * K, D), lambda i: (i, 0)),
            pl.BlockSpec((BB, O), lambda i: (i, 0)),
            pl.BlockSpec((D, D), lambda i: (0, 0)),
            pl.BlockSpec((1, D), lambda i: (0, 0)),
        ],
        out_shape=[
            jax.ShapeDtypeStruct((M, D), jnp.float32),
            jax.ShapeDtypeStruct((B, O), jnp.float32),
            jax.ShapeDtypeStruct((D, D), jnp.float32),
            jax.ShapeDtypeStruct((1, D), jnp.float32),
        ],
    )(scores, base_features, weight_1, b1, dlp_lin1_w, dlpb)

    out = pl.pallas_call(
        _k2_body,
        grid=(NBLK,),
        in_specs=[
            pl.BlockSpec((BB * K, D), lambda i: (i, 0)),
            pl.BlockSpec((BB, O), lambda i: (i, 0)),
            pl.BlockSpec((D, D), lambda i: (0, 0)),
            pl.BlockSpec((1, D), lambda i: (0, 0)),
            pl.BlockSpec((H, D), lambda i: (0, 0)),
            pl.BlockSpec((H, D), lambda i: (0, 0)),
            pl.BlockSpec((1, H), lambda i: (0, 0)),
            pl.BlockSpec((1, H), lambda i: (0, 0)),
            pl.BlockSpec((1, H), lambda i: (0, 0)),
            pl.BlockSpec((O, H), lambda i: (0, 0)),
            pl.BlockSpec((1, O), lambda i: (0, 0)),
        ],
        out_specs=pl.BlockSpec((BB, O), lambda i: (i, 0)),
        out_shape=jax.ShapeDtypeStruct((B, O), jnp.float32),
        scratch_shapes=[
            pltpu.VMEM((1, H), jnp.float32),
            pltpu.VMEM((1, H), jnp.float32),
        ],
    )(bf, newbase, c_mat, s_vec, mlp_w0, mlp_w0.astype(jnp.bfloat16), b0, g0,
      beta0, mlp_w1.astype(jnp.bfloat16), b1v)

    return out.astype(jnp.float32)


# speedup vs baseline: 1.0610x; 1.0610x over previous
import functools

import jax
import jax.numpy as jnp
from jax.experimental import pallas as pl
from jax.experimental.pallas import tpu as pltpu

B, N, D = 128, 193, 512
K = 96
H = 1024
O = 2048
BB = 8
NBLK = B // BB
BB2 = 16
NBLK2 = B // BB2
M = B * K

_HI = jax.lax.Precision.HIGHEST
_MED = jax.lax.Precision.DEFAULT


def _k1_body(scores_ref, x_ref, w1_ref, b1_ref, dlpw_ref, dlpb_ref,
             bf_ref, newbase_ref, c_ref, s_ref):
    i = pl.program_id(0)
    s = scores_ref[...]
    col = jax.lax.broadcasted_iota(jnp.int32, (BB, N), 1)
    s = jnp.where(col == 0, -1.0, s)

    si = s[:, :, None]
    sj = s[:, None, :]
    ii = jax.lax.broadcasted_iota(jnp.int32, (N, N), 0)[None]
    jj = jax.lax.broadcasted_iota(jnp.int32, (N, N), 1)[None]
    cmp = (sj > si) | ((sj == si) & (jj < ii))
    rank = jnp.sum(cmp.astype(jnp.float32), axis=2)

    r_iota = jax.lax.broadcasted_iota(jnp.int32, (BB, N, K), 2).astype(jnp.float32)
    p = (rank[:, :, None] == r_iota).astype(jnp.float32)

    x = x_ref[...]
    gathered = jax.lax.dot_general(
        p, x, (((1,), (1,)), ((0,), (0,))),
        preferred_element_type=jnp.float32, precision=_MED)

    w1v = w1_ref[...].reshape(1, 1, D)
    xs = jnp.sum(gathered * w1v, axis=2) + b1_ref[...]
    new = jax.lax.dot_general(
        xs, dlpw_ref[...], (((1,), (1,)), ((), ())),
        preferred_element_type=jnp.float32, precision=_MED) + dlpb_ref[...]
    nb = new * (1.0 / (jnp.sqrt(jnp.sum(new * new, axis=1, keepdims=True)) + 1e-8))
    newbase_ref[...] = nb

    sq = jnp.sum(gathered * gathered, axis=2, keepdims=True)
    bf = gathered * (1.0 / (jnp.sqrt(sq) + 1e-8))
    bf2 = bf.reshape(BB * K, D)
    bf_ref[...] = bf2.astype(jnp.bfloat16)

    cblk = jax.lax.dot_general(
        bf2, bf2, (((0,), (0,)), ((), ())),
        preferred_element_type=jnp.float32, precision=_MED)
    sblk = jnp.sum(bf2, axis=0, keepdims=True)

    @pl.when(i == 0)
    def _():
        c_ref[...] = cblk
        s_ref[...] = sblk

    @pl.when(i != 0)
    def _():
        c_ref[...] += cblk
        s_ref[...] += sblk


def _k2_body(bf_ref, newbase_ref, c_ref, s_ref, w0_ref, b0_ref, g0_ref,
             beta0_ref, w1b_ref, b1_ref, out_ref, w0s_ref, shift_ref):
    i = pl.program_id(0)

    @pl.when(i == 0)
    def _():
        w0 = w0_ref[...]
        sbar = s_ref[...] * (1.0 / M)
        m1 = jax.lax.dot_general(
            sbar, w0, (((1,), (1,)), ((), ())),
            preferred_element_type=jnp.float32, precision=_HI)
        w0c = jax.lax.dot_general(
            w0, c_ref[...], (((1,), (0,)), ((), ())),
            preferred_element_type=jnp.float32, precision=_HI)
        q = jnp.sum(w0c * w0, axis=1, keepdims=True).reshape(1, H) * (1.0 / M)
        b0 = b0_ref[...]
        mu = m1 + b0
        eh2 = q + 2.0 * b0 * m1 + b0 * b0
        var = eh2 - mu * mu
        sc = g0_ref[...] * jax.lax.rsqrt(var + 1e-5)
        w0s_ref[...] = (w0 * sc.reshape(H, 1)).astype(jnp.bfloat16)
        shift_ref[...] = (b0 - mu) * sc + beta0_ref[...]

    bfb = bf_ref[...]
    h = jax.lax.dot_general(
        bfb, w0s_ref[...], (((1,), (1,)), ((), ())),
        preferred_element_type=jnp.float32, precision=_MED)
    a = jnp.maximum(h + shift_ref[...], 0.0).astype(jnp.bfloat16)
    out = jax.lax.dot_general(
        a, w1b_ref[...], (((1,), (1,)), ((), ())),
        preferred_element_type=jnp.float32, precision=_MED)
    pooled = jnp.max(out.reshape(BB2, K, O), axis=1)
    out_ref[...] = pooled + b1_ref[...] + newbase_ref[...]


@functools.partial(jax.jit, static_argnames=())
def kernel(base_features, atten, pid, weight_1, bias_1, dlp_lin1_w, dlp_lin1_b,
           mlp_w0, mlp_b0, mlp_g0, mlp_beta0, mlp_w1, mlp_b1):
    del pid
    scores = atten[:, 0, :]

    b1 = bias_1.reshape(1, 1)
    dlpb = dlp_lin1_b.reshape(1, O)
    b0 = mlp_b0.reshape(1, H)
    g0 = mlp_g0.reshape(1, H)
    beta0 = mlp_beta0.reshape(1, H)
    b1v = mlp_b1.reshape(1, O)

    bf, newbase, c_mat, s_vec = pl.pallas_call(
        _k1_body,
        grid=(NBLK,),
        in_specs=[
            pl.BlockSpec((BB, N), lambda i: (i, 0)),
            pl.BlockSpec((BB, N, D), lambda i: (i, 0, 0)),
            pl.BlockSpec((1, D), lambda i: (0, 0)),
            pl.BlockSpec((1, 1), lambda i: (0, 0)),
            pl.BlockSpec((O, K), lambda i: (0, 0)),
            pl.BlockSpec((1, O), lambda i: (0, 0)),
        ],
        out_specs=[
            pl.BlockSpec((BB * K, D), lambda i: (i, 0)),
            pl.BlockSpec((BB, O), lambda i: (i, 0)),
            pl.BlockSpec((D, D), lambda i: (0, 0)),
            pl.BlockSpec((1, D), lambda i: (0, 0)),
        ],
        out_shape=[
            jax.ShapeDtypeStruct((M, D), jnp.bfloat16),
            jax.ShapeDtypeStruct((B, O), jnp.float32),
            jax.ShapeDtypeStruct((D, D), jnp.float32),
            jax.ShapeDtypeStruct((1, D), jnp.float32),
        ],
    )(scores, base_features, weight_1, b1, dlp_lin1_w, dlpb)

    out = pl.pallas_call(
        _k2_body,
        grid=(NBLK2,),
        in_specs=[
            pl.BlockSpec((BB2 * K, D), lambda i: (i, 0)),
            pl.BlockSpec((BB2, O), lambda i: (i, 0)),
            pl.BlockSpec((D, D), lambda i: (0, 0)),
            pl.BlockSpec((1, D), lambda i: (0, 0)),
            pl.BlockSpec((H, D), lambda i: (0, 0)),
            pl.BlockSpec((1, H), lambda i: (0, 0)),
            pl.BlockSpec((1, H), lambda i: (0, 0)),
            pl.BlockSpec((1, H), lambda i: (0, 0)),
            pl.BlockSpec((O, H), lambda i: (0, 0)),
            pl.BlockSpec((1, O), lambda i: (0, 0)),
        ],
        out_specs=pl.BlockSpec((BB2, O), lambda i: (i, 0)),
        out_shape=jax.ShapeDtypeStruct((B, O), jnp.float32),
        scratch_shapes=[
            pltpu.VMEM((H, D), jnp.bfloat16),
            pltpu.VMEM((1, H), jnp.float32),
        ],
    )(bf, newbase, c_mat, s_vec, mlp_w0, b0, g0,
      beta0, mlp_w1.astype(jnp.bfloat16), b1v)

    return out.astype(jnp.float32)


# speedup vs baseline: 1.0764x; 1.0145x over previous
import functools

import jax
import jax.numpy as jnp
from jax.experimental import pallas as pl
from jax.experimental.pallas import tpu as pltpu

B, N, D = 128, 193, 512
K = 96
H = 1024
O = 2048
BB = 8
NBLK = B // BB
BB2 = 16
NBLK2 = B // BB2
M = B * K

_HI = jax.lax.Precision.HIGHEST
_MED = jax.lax.Precision.DEFAULT


def _k1_body(scores_ref, x_ref, w1_ref, b1_ref, dlpw_ref, dlpb_ref,
             bf_ref, newbase_ref, c_ref, s_ref):
    i = pl.program_id(0)
    s = scores_ref[...]
    col = jax.lax.broadcasted_iota(jnp.int32, (BB, N), 1)
    s = jnp.where(col == 0, -1.0, s)

    si = s[:, :, None]
    sj = s[:, None, :]
    ii = jax.lax.broadcasted_iota(jnp.int32, (N, N), 0)[None]
    jj = jax.lax.broadcasted_iota(jnp.int32, (N, N), 1)[None]
    cmp = (sj > si) | ((sj == si) & (jj < ii))
    rank = jnp.sum(cmp.astype(jnp.float32), axis=2)

    r_iota = jax.lax.broadcasted_iota(jnp.int32, (BB, N, K), 2).astype(jnp.float32)
    p = (rank[:, :, None] == r_iota).astype(jnp.float32)

    x = x_ref[...]
    gathered = jax.lax.dot_general(
        p, x, (((1,), (1,)), ((0,), (0,))),
        preferred_element_type=jnp.float32, precision=_MED)

    w1v = w1_ref[...].reshape(1, 1, D)
    xs = jnp.sum(gathered * w1v, axis=2) + b1_ref[...]
    new = jax.lax.dot_general(
        xs, dlpw_ref[...], (((1,), (1,)), ((), ())),
        preferred_element_type=jnp.float32, precision=_MED) + dlpb_ref[...]
    nb = new * (1.0 / (jnp.sqrt(jnp.sum(new * new, axis=1, keepdims=True)) + 1e-8))
    newbase_ref[...] = nb

    sq = jnp.sum(gathered * gathered, axis=2, keepdims=True)
    bf = gathered * (1.0 / (jnp.sqrt(sq) + 1e-8))
    bf2 = bf.reshape(BB * K, D)
    bf_ref[...] = bf2.astype(jnp.bfloat16)

    cblk = jax.lax.dot_general(
        bf2, bf2, (((0,), (0,)), ((), ())),
        preferred_element_type=jnp.float32, precision=_MED)
    sblk = jnp.sum(bf2, axis=0, keepdims=True)

    @pl.when(i == 0)
    def _():
        c_ref[...] = cblk
        s_ref[...] = sblk

    @pl.when(i != 0)
    def _():
        c_ref[...] += cblk
        s_ref[...] += sblk


def _k2_body(bf_ref, newbase_ref, c_ref, s_ref, w0_ref, b0_ref, g0_ref,
             beta0_ref, w1_ref, b1_ref, out_ref, w0s_ref, w1s_ref, shift_ref):
    i = pl.program_id(0)

    @pl.when(i == 0)
    def _():
        w1s_ref[...] = w1_ref[...].astype(jnp.bfloat16)
        w0 = w0_ref[...]
        sbar = s_ref[...] * (1.0 / M)
        m1 = jax.lax.dot_general(
            sbar, w0, (((1,), (1,)), ((), ())),
            preferred_element_type=jnp.float32, precision=_HI)
        w0c = jax.lax.dot_general(
            w0, c_ref[...], (((1,), (0,)), ((), ())),
            preferred_element_type=jnp.float32, precision=_HI)
        q = jnp.sum(w0c * w0, axis=1, keepdims=True).reshape(1, H) * (1.0 / M)
        b0 = b0_ref[...]
        mu = m1 + b0
        eh2 = q + 2.0 * b0 * m1 + b0 * b0
        var = eh2 - mu * mu
        sc = g0_ref[...] * jax.lax.rsqrt(var + 1e-5)
        w0s_ref[...] = (w0 * sc.reshape(H, 1)).astype(jnp.bfloat16)
        shift_ref[...] = (b0 - mu) * sc + beta0_ref[...]

    bfb = bf_ref[...]
    HC = H // 2
    a_parts = []
    for c in range(2):
        hc = jax.lax.dot_general(
            bfb, w0s_ref[c * HC:(c + 1) * HC, :], (((1,), (1,)), ((), ())),
            preferred_element_type=jnp.float32, precision=_MED)
        a_parts.append(jnp.maximum(hc + shift_ref[:, c * HC:(c + 1) * HC],
                                   0.0).astype(jnp.bfloat16))
    a = jnp.concatenate(a_parts, axis=1)
    OC = O // 4
    for c in range(4):
        outc = jax.lax.dot_general(
            a, w1s_ref[c * OC:(c + 1) * OC, :], (((1,), (1,)), ((), ())),
            preferred_element_type=jnp.float32, precision=_MED)
        pooled = jnp.max(outc.reshape(BB2, K, OC), axis=1)
        out_ref[:, c * OC:(c + 1) * OC] = (
            pooled + b1_ref[:, c * OC:(c + 1) * OC]
            + newbase_ref[:, c * OC:(c + 1) * OC])


@functools.partial(jax.jit, static_argnames=())
def kernel(base_features, atten, pid, weight_1, bias_1, dlp_lin1_w, dlp_lin1_b,
           mlp_w0, mlp_b0, mlp_g0, mlp_beta0, mlp_w1, mlp_b1):
    del pid
    scores = atten[:, 0, :]

    b1 = bias_1.reshape(1, 1)
    dlpb = dlp_lin1_b.reshape(1, O)
    b0 = mlp_b0.reshape(1, H)
    g0 = mlp_g0.reshape(1, H)
    beta0 = mlp_beta0.reshape(1, H)
    b1v = mlp_b1.reshape(1, O)

    bf, newbase, c_mat, s_vec = pl.pallas_call(
        _k1_body,
        grid=(NBLK,),
        in_specs=[
            pl.BlockSpec((BB, N), lambda i: (i, 0)),
            pl.BlockSpec((BB, N, D), lambda i: (i, 0, 0)),
            pl.BlockSpec((1, D), lambda i: (0, 0)),
            pl.BlockSpec((1, 1), lambda i: (0, 0)),
            pl.BlockSpec((O, K), lambda i: (0, 0)),
            pl.BlockSpec((1, O), lambda i: (0, 0)),
        ],
        out_specs=[
            pl.BlockSpec((BB * K, D), lambda i: (i, 0)),
            pl.BlockSpec((BB, O), lambda i: (i, 0)),
            pl.BlockSpec((D, D), lambda i: (0, 0)),
            pl.BlockSpec((1, D), lambda i: (0, 0)),
        ],
        out_shape=[
            jax.ShapeDtypeStruct((M, D), jnp.bfloat16),
            jax.ShapeDtypeStruct((B, O), jnp.float32),
            jax.ShapeDtypeStruct((D, D), jnp.float32),
            jax.ShapeDtypeStruct((1, D), jnp.float32),
        ],
    )(scores, base_features, weight_1, b1, dlp_lin1_w, dlpb)

    out = pl.pallas_call(
        _k2_body,
        grid=(NBLK2,),
        in_specs=[
            pl.BlockSpec((BB2 * K, D), lambda i: (i, 0)),
            pl.BlockSpec((BB2, O), lambda i: (i, 0)),
            pl.BlockSpec((D, D), lambda i: (0, 0)),
            pl.BlockSpec((1, D), lambda i: (0, 0)),
            pl.BlockSpec((H, D), lambda i: (0, 0)),
            pl.BlockSpec((1, H), lambda i: (0, 0)),
            pl.BlockSpec((1, H), lambda i: (0, 0)),
            pl.BlockSpec((1, H), lambda i: (0, 0)),
            pl.BlockSpec((O, H), lambda i: (0, 0)),
            pl.BlockSpec((1, O), lambda i: (0, 0)),
        ],
        out_specs=pl.BlockSpec((BB2, O), lambda i: (i, 0)),
        out_shape=jax.ShapeDtypeStruct((B, O), jnp.float32),
        scratch_shapes=[
            pltpu.VMEM((H, D), jnp.bfloat16),
            pltpu.VMEM((O, H), jnp.bfloat16),
            pltpu.VMEM((1, H), jnp.float32),
        ],
    )(bf, newbase, c_mat, s_vec, mlp_w0, b0, g0, beta0, mlp_w1, b1v)

    return out.astype(jnp.float32)
